# baseline (speedup 1.0000x reference)
import functools

import jax
import jax.numpy as jnp
from jax import lax
from jax.experimental import pallas as pl
from jax.experimental.pallas import tpu as pltpu
from jax.experimental.pallas import tpu_sc as plsc

N_EMB = 1024
EMB_DIM = 64
BETA = 0.25
N_ROWS = 8192
HALF = N_ROWS // 2
BLK = 512
HGRID = HALF // BLK

SC_NC = 2
SC_NS = 16
SC_NW = SC_NC * SC_NS
ROWS_PER_W = HALF // SC_NW


def _sc_gather_body(idx_hbm, emb_hbm, out_hbm, idx_v, rows_v, sem):
    wid = lax.axis_index("s") * SC_NC + lax.axis_index("c")
    base = wid * ROWS_PER_W
    pltpu.sync_copy(idx_hbm.at[pl.ds(base, ROWS_PER_W)], idx_v)
    pltpu.async_copy(emb_hbm.at[idx_v], rows_v, sem).wait()
    pltpu.sync_copy(rows_v, out_hbm.at[pl.ds(base, ROWS_PER_W)])


_sc_gather = functools.partial(
    pl.kernel,
    mesh=plsc.VectorSubcoreMesh(core_axis_name="c", subcore_axis_name="s"),
    out_type=jax.ShapeDtypeStruct((HALF, 128), jnp.float32),
    scratch_types=[
        pltpu.VMEM((ROWS_PER_W,), jnp.int32),
        pltpu.VMEM((ROWS_PER_W, 128), jnp.float32),
        pltpu.SemaphoreType.DMA,
    ],
)(_sc_gather_body)


def _dist_block(z, et):
    z2 = jnp.sum(z * z, axis=1, keepdims=True)
    e2 = jnp.sum(et * et, axis=0, keepdims=True)
    d = (z2 + e2) - 2.0 * jnp.dot(z, et, preferred_element_type=jnp.float32)
    mind = jnp.min(d, axis=1)
    iota = jax.lax.broadcasted_iota(jnp.int32, (BLK, N_EMB), 1)
    idx = jnp.min(jnp.where(d == mind[:, None], iota, N_EMB),
                  axis=1).astype(jnp.int32)
    onehot = (iota == idx[:, None]).astype(jnp.float32)
    cnt = jnp.sum(onehot, axis=0, keepdims=True)
    return d, mind, idx, onehot, cnt


def _tc_body_a(z_ref, embt_ref,
               dist_ref, idx_ref, counts_ref, minsum_out_ref,
               minsum_ref):
    i = pl.program_id(0)
    d, mind, idx, _, cnt = _dist_block(z_ref[...], embt_ref[...])
    dist_ref[...] = d
    idx_ref[...] = idx.reshape(1, 1, BLK)

    @pl.when(i == 0)
    def _init():
        counts_ref[...] = jnp.zeros((1, N_EMB), jnp.float32)
        minsum_ref[0, 0] = 0.0

    counts_ref[...] += cnt
    minsum_ref[0, 0] += jnp.sum(mind)

    @pl.when(i == HGRID - 1)
    def _final():
        minsum_out_ref[...] = jnp.full((1, 1), minsum_ref[0, 0], jnp.float32)


def _tc_body_b(z_ref, embt_ref, emb_ref, cin_ref, minin_ref, dist_in_ref,
               dist_ref, idx_ref, zq_ref, loss_ref, perp_ref,
               counts_ref, minsum_ref):
    i = pl.program_id(0)
    d, mind, idx, onehot, cnt = _dist_block(z_ref[...], embt_ref[...])
    dist_ref[...] = d
    idx_ref[...] = idx.reshape(1, 1, BLK)
    zq_ref[...] = jnp.dot(onehot, emb_ref[...],
                          preferred_element_type=jnp.float32)

    @pl.when(i == 0)
    def _init():
        counts_ref[...] = cin_ref[...]
        minsum_ref[0, 0] = minin_ref[0, 0]

    counts_ref[...] += cnt
    minsum_ref[0, 0] += jnp.sum(mind)

    @pl.when(i == HGRID - 1)
    def _final():
        loss = BETA * minsum_ref[0, 0] / float(N_ROWS * EMB_DIM)
        loss_ref[...] = jnp.full((1, 1), loss, jnp.float32)
        p = counts_ref[...] / float(N_ROWS)
        perp = jnp.exp(-jnp.sum(p * jnp.log(p + 1e-10)))
        perp_ref[...] = jnp.full((1, 1), perp, jnp.float32)


def kernel(z_e, embedding):
    B, D, H, W = z_e.shape
    z_flat = jnp.transpose(z_e, (0, 2, 3, 1)).reshape(N_ROWS, EMB_DIM)
    emb_t = embedding.T

    z_a = z_flat[:HALF]
    z_b = z_flat[HALF:]

    dist_a, idx_a3, counts_a, minsum_a = pl.pallas_call(
        _tc_body_a,
        grid=(HGRID,),
        in_specs=[
            pl.BlockSpec((BLK, EMB_DIM), lambda i: (i, 0)),
            pl.BlockSpec((EMB_DIM, N_EMB), lambda i: (0, 0)),
        ],
        out_specs=(
            pl.BlockSpec((BLK, N_EMB), lambda i: (i, 0)),
            pl.BlockSpec((1, 1, BLK), lambda i: (i, 0, 0)),
            pl.BlockSpec((1, N_EMB), lambda i: (0, 0)),
            pl.BlockSpec((1, 1), lambda i: (0, 0)),
        ),
        out_shape=(
            jax.ShapeDtypeStruct((N_ROWS, N_EMB), jnp.float32),
            jax.ShapeDtypeStruct((HGRID, 1, BLK), jnp.int32),
            jax.ShapeDtypeStruct((1, N_EMB), jnp.float32),
            jax.ShapeDtypeStruct((1, 1), jnp.float32),
        ),
        scratch_shapes=[pltpu.SMEM((1, 1), jnp.float32)],
    )(z_a, emb_t)

    idx_a = idx_a3.reshape(HALF)
    emb_pad = jnp.pad(embedding, ((0, 0), (0, 128 - EMB_DIM)))
    zq_a = _sc_gather(idx_a, emb_pad)

    dist, idx_b3, zq_b, loss, perp = pl.pallas_call(
        _tc_body_b,
        grid=(HGRID,),
        in_specs=[
            pl.BlockSpec((BLK, EMB_DIM), lambda i: (i, 0)),
            pl.BlockSpec((EMB_DIM, N_EMB), lambda i: (0, 0)),
            pl.BlockSpec((N_EMB, EMB_DIM), lambda i: (0, 0)),
            pl.BlockSpec((1, N_EMB), lambda i: (0, 0)),
            pl.BlockSpec((1, 1), lambda i: (0, 0)),
            pl.BlockSpec(memory_space=pl.ANY),
        ],
        out_specs=(
            pl.BlockSpec((BLK, N_EMB), lambda i: (i + HGRID, 0)),
            pl.BlockSpec((1, 1, BLK), lambda i: (i, 0, 0)),
            pl.BlockSpec((BLK, EMB_DIM), lambda i: (i, 0)),
            pl.BlockSpec((1, 1), lambda i: (0, 0)),
            pl.BlockSpec((1, 1), lambda i: (0, 0)),
        ),
        out_shape=(
            jax.ShapeDtypeStruct((N_ROWS, N_EMB), jnp.float32),
            jax.ShapeDtypeStruct((HGRID, 1, BLK), jnp.int32),
            jax.ShapeDtypeStruct((HALF, EMB_DIM), jnp.float32),
            jax.ShapeDtypeStruct((1, 1), jnp.float32),
            jax.ShapeDtypeStruct((1, 1), jnp.float32),
        ),
        scratch_shapes=[pltpu.VMEM((1, N_EMB), jnp.float32),
                        pltpu.SMEM((1, 1), jnp.float32)],
        input_output_aliases={5: 0},
    )(z_b, emb_t, embedding, counts_a, minsum_a, dist_a)

    idx_b = idx_b3.reshape(HALF)
    encoding_indices = jnp.concatenate([idx_a, idx_b])
    zq = jnp.concatenate([zq_a[:, :EMB_DIM], zq_b], axis=0)
    z_q_out = jnp.transpose(zq.reshape(B, H, W, D), (0, 3, 1, 2))
    return (z_q_out, loss.reshape(()), perp.reshape(()),
            encoding_indices, dist)


# speedup vs baseline: 1.8874x; 1.8874x over previous
import jax
import jax.numpy as jnp
from jax.experimental import pallas as pl
from jax.experimental.pallas import tpu as pltpu

N_EMB = 1024
EMB_DIM = 64
BETA = 0.25
N_ROWS = 8192
BLK = 512
GRID = N_ROWS // BLK


def _tc_body(z_ref, embt_ref, emb_ref,
             dist_ref, idx_ref, zq_ref, counts_ref, loss_ref, perp_ref,
             minsum_ref):
    i = pl.program_id(0)
    z = z_ref[...]
    et = embt_ref[...]
    z2 = jnp.sum(z * z, axis=1, keepdims=True)
    e2 = jnp.sum(et * et, axis=0, keepdims=True)
    d = (z2 + e2) - jnp.dot(z + z, et, preferred_element_type=jnp.float32)
    dist_ref[...] = d

    mind = jnp.min(d, axis=1)
    iota = jax.lax.broadcasted_iota(jnp.int32, (BLK, N_EMB), 1)
    idx = jnp.min(jnp.where(d == mind[:, None], iota, N_EMB),
                  axis=1).astype(jnp.int32)
    idx_ref[...] = idx.reshape(1, 1, BLK)

    onehot = (iota == idx[:, None]).astype(jnp.float32)
    zq_ref[...] = jnp.dot(onehot, emb_ref[...],
                          preferred_element_type=jnp.float32)
    cnt = jnp.sum(onehot, axis=0, keepdims=True)

    @pl.when(i == 0)
    def _init():
        counts_ref[...] = jnp.zeros((1, N_EMB), jnp.float32)
        minsum_ref[0, 0] = 0.0

    counts_ref[...] += cnt
    minsum_ref[0, 0] += jnp.sum(mind)

    @pl.when(i == GRID - 1)
    def _final():
        loss = BETA * minsum_ref[0, 0] / float(N_ROWS * EMB_DIM)
        loss_ref[...] = jnp.full((1, 1), loss, jnp.float32)
        p = counts_ref[...] / float(N_ROWS)
        perp = jnp.exp(-jnp.sum(p * jnp.log(p + 1e-10)))
        perp_ref[...] = jnp.full((1, 1), perp, jnp.float32)


def kernel(z_e, embedding):
    B, D, H, W = z_e.shape
    z_flat = jnp.transpose(z_e, (0, 2, 3, 1)).reshape(N_ROWS, EMB_DIM)
    emb_t = embedding.T

    out_shapes = (
        jax.ShapeDtypeStruct((N_ROWS, N_EMB), jnp.float32),
        jax.ShapeDtypeStruct((GRID, 1, BLK), jnp.int32),
        jax.ShapeDtypeStruct((N_ROWS, EMB_DIM), jnp.float32),
        jax.ShapeDtypeStruct((1, N_EMB), jnp.float32),
        jax.ShapeDtypeStruct((1, 1), jnp.float32),
        jax.ShapeDtypeStruct((1, 1), jnp.float32),
    )
    dist, idx3, zq, counts, loss, perp = pl.pallas_call(
        _tc_body,
        grid=(GRID,),
        in_specs=[
            pl.BlockSpec((BLK, EMB_DIM), lambda i: (i, 0)),
            pl.BlockSpec((EMB_DIM, N_EMB), lambda i: (0, 0)),
            pl.BlockSpec((N_EMB, EMB_DIM), lambda i: (0, 0)),
        ],
        out_specs=(
            pl.BlockSpec((BLK, N_EMB), lambda i: (i, 0)),
            pl.BlockSpec((1, 1, BLK), lambda i: (i, 0, 0)),
            pl.BlockSpec((BLK, EMB_DIM), lambda i: (i, 0)),
            pl.BlockSpec((1, N_EMB), lambda i: (0, 0)),
            pl.BlockSpec((1, 1), lambda i: (0, 0)),
            pl.BlockSpec((1, 1), lambda i: (0, 0)),
        ),
        out_shape=out_shapes,
        scratch_shapes=[pltpu.SMEM((1, 1), jnp.float32)],
    )(z_flat, emb_t, embedding)

    encoding_indices = idx3.reshape(N_ROWS)
    z_q_out = jnp.transpose(zq.reshape(B, H, W, D), (0, 3, 1, 2))
    return (z_q_out, loss.reshape(()), perp.reshape(()),
            encoding_indices, dist)


# speedup vs baseline: 2.2236x; 1.1781x over previous
import jax
import jax.numpy as jnp
from jax.experimental import pallas as pl
from jax.experimental.pallas import tpu as pltpu

N_EMB = 1024
EMB_DIM = 64
BETA = 0.25
N_ROWS = 8192
BLK = 1024
GRID = N_ROWS // BLK


def _tc_body(z_ref, embt_ref, emb_ref,
             dist_ref, idx_ref, zq_ref, counts_ref, loss_ref, perp_ref,
             minsum_ref):
    i = pl.program_id(0)
    z = z_ref[...]
    et = embt_ref[...]
    z2 = jnp.sum(z * z, axis=1, keepdims=True)
    e2 = jnp.sum(et * et, axis=0, keepdims=True)
    d = (z2 + e2) - jnp.dot(z + z, et, preferred_element_type=jnp.float32)
    dist_ref[...] = d

    mind = jnp.min(d, axis=1)
    iotaf = jax.lax.broadcasted_iota(
        jnp.int32, (BLK, N_EMB), 1).astype(jnp.float32)
    idxf = jnp.min(jnp.where(d == mind[:, None], iotaf, float(N_EMB)),
                   axis=1)
    idx = idxf.astype(jnp.int32)
    idx_ref[...] = idx.reshape(1, 1, BLK)

    onehot = (iotaf == idxf[:, None]).astype(jnp.float32)
    zq_ref[...] = jnp.dot(onehot, emb_ref[...],
                          preferred_element_type=jnp.float32)
    cnt = jnp.sum(onehot, axis=0, keepdims=True)

    @pl.when(i == 0)
    def _init():
        counts_ref[...] = jnp.zeros((1, N_EMB), jnp.float32)
        minsum_ref[0, 0] = 0.0

    counts_ref[...] += cnt
    minsum_ref[0, 0] += jnp.sum(mind)

    @pl.when(i == GRID - 1)
    def _final():
        loss = BETA * minsum_ref[0, 0] / float(N_ROWS * EMB_DIM)
        loss_ref[...] = jnp.full((1, 1), loss, jnp.float32)
        p = counts_ref[...] / float(N_ROWS)
        perp = jnp.exp(-jnp.sum(p * jnp.log(p + 1e-10)))
        perp_ref[...] = jnp.full((1, 1), perp, jnp.float32)


def kernel(z_e, embedding):
    B, D, H, W = z_e.shape
    z_flat = jnp.transpose(z_e, (0, 2, 3, 1)).reshape(N_ROWS, EMB_DIM)
    emb_t = embedding.T

    out_shapes = (
        jax.ShapeDtypeStruct((N_ROWS, N_EMB), jnp.float32),
        jax.ShapeDtypeStruct((GRID, 1, BLK), jnp.int32),
        jax.ShapeDtypeStruct((N_ROWS, EMB_DIM), jnp.float32),
        jax.ShapeDtypeStruct((1, N_EMB), jnp.float32),
        jax.ShapeDtypeStruct((1, 1), jnp.float32),
        jax.ShapeDtypeStruct((1, 1), jnp.float32),
    )
    dist, idx3, zq, counts, loss, perp = pl.pallas_call(
        _tc_body,
        grid=(GRID,),
        in_specs=[
            pl.BlockSpec((BLK, EMB_DIM), lambda i: (i, 0)),
            pl.BlockSpec((EMB_DIM, N_EMB), lambda i: (0, 0)),
            pl.BlockSpec((N_EMB, EMB_DIM), lambda i: (0, 0)),
        ],
        out_specs=(
            pl.BlockSpec((BLK, N_EMB), lambda i: (i, 0)),
            pl.BlockSpec((1, 1, BLK), lambda i: (i, 0, 0)),
            pl.BlockSpec((BLK, EMB_DIM), lambda i: (i, 0)),
            pl.BlockSpec((1, N_EMB), lambda i: (0, 0)),
            pl.BlockSpec((1, 1), lambda i: (0, 0)),
            pl.BlockSpec((1, 1), lambda i: (0, 0)),
        ),
        out_shape=out_shapes,
        scratch_shapes=[pltpu.SMEM((1, 1), jnp.float32)],
    )(z_flat, emb_t, embedding)

    encoding_indices = idx3.reshape(N_ROWS)
    z_q_out = jnp.transpose(zq.reshape(B, H, W, D), (0, 3, 1, 2))
    return (z_q_out, loss.reshape(()), perp.reshape(()),
            encoding_indices, dist)
